# Initial kernel scaffold; baseline (speedup 1.0000x reference)
import jax
import jax.numpy as jnp
from jax.experimental import pallas as pl


def kernel(x, edge_index_1, edge_index_2, edge_index_n, num_graphs, conv0, conv1_1, conv1_2, conv2, conv3, conv4, lin1, lin2, lin3, lin4):
    raise NotImplementedError("write your pallas kernel here")


# speedup vs baseline: 1.0747x; 1.0747x over previous
import jax
import jax.numpy as jnp
from jax.experimental import pallas as pl

_N = 10000
_NG = 10


def _mm_kernel(x_ref, w_ref, o_ref):
    o_ref[...] = jnp.dot(x_ref[...], w_ref[...],
                         preferred_element_type=jnp.float32)


def _mm(x, w, rows=1000):
    m, k = x.shape
    n = w.shape[1]
    return pl.pallas_call(
        _mm_kernel,
        grid=(m // rows,),
        in_specs=[pl.BlockSpec((rows, k), lambda i: (i, 0)),
                  pl.BlockSpec((k, n), lambda i: (0, 0))],
        out_specs=pl.BlockSpec((rows, n), lambda i: (i, 0)),
        out_shape=jax.ShapeDtypeStruct((m, n), jnp.float32),
    )(x, w)


def _gat(x, ei, p):
    W, a_s, a_d, b = p
    n = x.shape[0]
    k = W.shape[0]
    w_ext = jnp.concatenate(
        [W, (W @ a_s)[:, None], (W @ a_d)[:, None],
         jnp.zeros((k, 126), jnp.float32)], axis=1)
    xe = _mm(x, w_ext)
    xp = xe[:, :512]
    s = xe[:, 512]
    d = xe[:, 513]
    loop = jnp.arange(n, dtype=ei.dtype)
    src = jnp.concatenate([ei[0], loop])
    dst = jnp.concatenate([ei[1], loop])
    e = jax.nn.leaky_relu(s[src] + d[dst], negative_slope=0.2)
    m = jax.ops.segment_max(e, dst, num_segments=n)
    ex = jnp.exp(e - m[dst])
    den = jax.ops.segment_sum(ex, dst, num_segments=n)
    alpha = ex / (den[dst] + 1e-16)
    out = jax.ops.segment_sum(alpha[:, None] * xp[src], dst, num_segments=n)
    return out + b


def _lin(x, p, rows=1000):
    w, b = p
    n = w.shape[1]
    if n % 128 != 0:
        w = jnp.concatenate(
            [w, jnp.zeros((w.shape[0], 128 - n % 128), jnp.float32)], axis=1)
        return _mm(x, w, rows)[:, :n] + b
    return _mm(x, w, rows) + b


def kernel(x, edge_index_1, edge_index_2, edge_index_n, num_graphs,
           conv0, conv1_1, conv1_2, conv2, conv3, conv4,
           lin1, lin2, lin3, lin4):
    n = x.shape[0]
    npg = n // _NG
    x = x * (jnp.asarray(num_graphs, dtype=x.dtype) / jnp.float32(_NG))
    out = _gat(x, edge_index_n, conv0)
    out_1 = _gat(out, edge_index_1, conv1_1)
    out_2 = _gat(out, edge_index_2, conv1_2)
    out_lin1 = _lin(out_1 + out_2, lin1)
    out_lin2 = _lin(out_lin1, lin2)
    out_lin3 = _lin(out_lin2, lin3) + out_lin1
    left = _gat(out_lin3, edge_index_1, conv2)
    left = _gat(left, edge_index_1, conv3)
    left = _gat(left, edge_index_n, conv4)
    left = jnp.squeeze(_lin(left, lin4).reshape(_NG, npg, -1), axis=-1)
    probs_left = jax.nn.softmax(left, axis=-1)
    right = _gat(out_lin3, edge_index_2, conv2)
    right = _gat(right, edge_index_2, conv3)
    right = _gat(right, edge_index_n, conv4)
    right = jnp.squeeze(_lin(right, lin4).reshape(_NG, npg, -1), axis=-1)
    probs_right = jax.nn.softmax(right, axis=-1)
    out_to_critic = out_lin3.reshape(_NG, npg, -1)
    return (probs_left, probs_right, out_to_critic)


# speedup vs baseline: 3.2320x; 3.0075x over previous
import functools

import jax
import jax.numpy as jnp
from jax import lax
from jax.experimental import pallas as pl
from jax.experimental.pallas import tpu as pltpu
from jax.experimental.pallas import tpu_sc as plsc

_N = 10000
_NG = 10
_E = 160000
_EL = _E + _N
_HALF = _N // 2
_P = 90112
_TCH = 88
_TE = _TCH * 64
_ROWS = 5120
_STRIPE = _ROWS // 16
_SCRATCH = 5000


def _mm_kernel(x_ref, w_ref, o_ref):
    o_ref[...] = jnp.dot(x_ref[...], w_ref[...],
                         preferred_element_type=jnp.float32)


def _mm(x, w, rows=1000):
    m, k = x.shape
    n = w.shape[1]
    return pl.pallas_call(
        _mm_kernel,
        grid=(m // rows,),
        in_specs=[pl.BlockSpec((rows, k), lambda i: (i, 0)),
                  pl.BlockSpec((k, n), lambda i: (0, 0))],
        out_specs=pl.BlockSpec((rows, n), lambda i: (i, 0)),
        out_shape=jax.ShapeDtypeStruct((m, n), jnp.float32),
    )(x, w)


def _edge_prep(ei):
    loop = jnp.arange(_N, dtype=ei.dtype)
    src = jnp.concatenate([ei[0], loop]).astype(jnp.int32)
    dst = jnp.concatenate([ei[1], loop]).astype(jnp.int32)
    side = (dst >= _HALF).astype(jnp.int32)
    perm = jnp.argsort(side, stable=True).astype(jnp.int32)
    p0 = perm[:_P]
    p1 = perm[_EL - _P:]
    v0 = side[p0] == 0
    v1 = side[p1] == 1
    src0 = jnp.where(v0, src[p0], 0)
    dl0 = jnp.where(v0, dst[p0], _SCRATCH)
    src1 = jnp.where(v1, src[p1], 0)
    dl1 = jnp.where(v1, dst[p1] - _HALF, _SCRATCH)
    srcs = jnp.stack([src0, src1]).reshape(2, _P // 64, 64)
    dls = jnp.stack([dl0, dl1]).reshape(2, _P // 64, 64)
    return srcs, dls


def _edge_body(xpq0, xpq1, xpq2, xpq3, s_hbm, d_hbm, srcs_hbm, dls_hbm,
               o0_hbm, o1_hbm, o2_hbm, o3_hbm,
               s_v, d_v, srcc_v, dlc_v, e_v, rows_v, den_v, red_v,
               z_v, m16_v, out_s, den_s, mst_s):
    c = lax.axis_index("c")
    sid = lax.axis_index("s")
    coff = c * _HALF

    pltpu.sync_copy(s_hbm, s_v)
    pltpu.sync_copy(d_hbm, d_v)
    pltpu.sync_copy(srcs_hbm.at[c, pl.ds(sid * _TCH, _TCH)], srcc_v)
    pltpu.sync_copy(dls_hbm.at[c, pl.ds(sid * _TCH, _TCH)], dlc_v)

    def p1_body(j, mx):
        def p1_inner(k, mx):
            si = srcc_v[j, pl.ds(k * 16, 16)]
            di = dlc_v[j, pl.ds(k * 16, 16)] + coff
            z = plsc.load_gather(s_v, [si]) + plsc.load_gather(d_v, [di])
            e = jnp.where(z > 0, z, 0.2 * z)
            e_v[pl.ds(j * 64 + k * 16, 16)] = e
            return jnp.maximum(mx, e)
        mx = p1_inner(0, mx)
        mx = p1_inner(1, mx)
        mx = p1_inner(2, mx)
        mx = p1_inner(3, mx)
        return mx
    mx = lax.fori_loop(0, _TCH, p1_body,
                       jnp.full((16,), -3e38, jnp.float32))
    red_v[...] = mx

    pltpu.sync_copy(red_v, mst_s.at[sid])
    plsc.subcore_barrier()
    def z_body(i, _):
        z_v[pl.ds(i * 16, 16)] = jnp.zeros((16,), jnp.float32)
        return 0
    lax.fori_loop(0, _STRIPE // 16, z_body, 0)
    pltpu.sync_copy(z_v, den_s.at[pl.ds(sid * _STRIPE, _STRIPE)])
    pltpu.sync_copy(mst_s, m16_v)
    mall = m16_v[0, pl.ds(0, 16)]
    for t in range(1, 16):
        mall = jnp.maximum(mall, m16_v[t, pl.ds(0, 16)])
    m = jnp.max(mall)
    mb = jnp.full((16,), m)

    def p2_body(i, _):
        e_v[pl.ds(i * 16, 16)] = jnp.exp(e_v[pl.ds(i * 16, 16)] - mb)
        return 0
    lax.fori_loop(0, _TE // 16, p2_body, 0)
    plsc.subcore_barrier()

    def den_body(j, _):
        pltpu.sync_copy(e_v.at[pl.ds(j * 64, 64)],
                        den_s.at[dlc_v.at[j]], add=True)
        return 0
    lax.fori_loop(0, _TCH, den_body, 0)
    plsc.subcore_barrier()
    pltpu.sync_copy(den_s, den_v)

    def p3_body(j, _):
        def p3_inner(k):
            dl = dlc_v[j, pl.ds(k * 16, 16)]
            dn = plsc.load_gather(den_v, [dl])
            sl = pl.ds(j * 64 + k * 16, 16)
            e_v[sl] = e_v[sl] / (dn + 1e-16)
        p3_inner(0)
        p3_inner(1)
        p3_inner(2)
        p3_inner(3)
        return 0
    lax.fori_loop(0, _TCH, p3_body, 0)

    for xp_hbm, o_hbm in ((xpq0, o0_hbm), (xpq1, o1_hbm),
                          (xpq2, o2_hbm), (xpq3, o3_hbm)):
        def zr_body(r, _):
            for ck in range(8):
                rows_v[r, pl.ds(ck * 16, 16)] = jnp.zeros((16,),
                                                          jnp.float32)
            return 0
        lax.fori_loop(0, 64, zr_body, 0)
        for q in range(_STRIPE // 64):
            pltpu.sync_copy(rows_v,
                            out_s.at[pl.ds(sid * _STRIPE + q * 64, 64)])
        plsc.subcore_barrier()

        def ch_body(j, _):
            pltpu.sync_copy(xp_hbm.at[srcc_v.at[j]], rows_v)
            for g in range(4):
                a16 = e_v[pl.ds(j * 64 + g * 16, 16)]
                for i2 in range(16):
                    a = jnp.full((16,), a16[i2])
                    r = g * 16 + i2
                    for ck in range(8):
                        sl = pl.ds(ck * 16, 16)
                        rows_v[r, sl] = rows_v[r, sl] * a
            pltpu.sync_copy(rows_v, out_s.at[dlc_v.at[j]], add=True)
            return 0
        lax.fori_loop(0, _TCH, ch_body, 0)
        plsc.subcore_barrier()

        for q in range(_STRIPE // 64):
            pltpu.sync_copy(out_s.at[pl.ds(sid * _STRIPE + q * 64, 64)],
                            rows_v)
            pltpu.sync_copy(
                rows_v,
                o_hbm.at[pl.ds(c * _ROWS + sid * _STRIPE + q * 64, 64)])
        plsc.subcore_barrier()


_EDGE_KERNEL = None


def _edge_kernel():
    global _EDGE_KERNEL
    if _EDGE_KERNEL is None:
        mesh = plsc.VectorSubcoreMesh(core_axis_name="c",
                                      subcore_axis_name="s")
        _EDGE_KERNEL = pl.kernel(
            _edge_body,
            mesh=mesh,
            compiler_params=pltpu.CompilerParams(
                needs_layout_passes=False),
            out_type=[
                jax.ShapeDtypeStruct((2 * _ROWS, 128), jnp.float32)
                for _ in range(4)
            ],
            scratch_types=[
                pltpu.VMEM((_N + 16,), jnp.float32),
                pltpu.VMEM((_N + 16,), jnp.float32),
                pltpu.VMEM((_TCH, 64), jnp.int32),
                pltpu.VMEM((_TCH, 64), jnp.int32),
                pltpu.VMEM((_TE,), jnp.float32),
                pltpu.VMEM((64, 128), jnp.float32),
                pltpu.VMEM((_ROWS,), jnp.float32),
                pltpu.VMEM((16,), jnp.float32),
                pltpu.VMEM((_STRIPE,), jnp.float32),
                pltpu.VMEM((16, 16), jnp.float32),
                pltpu.VMEM_SHARED((_ROWS, 128), jnp.float32),
                pltpu.VMEM_SHARED((_ROWS,), jnp.float32),
                pltpu.VMEM_SHARED((16, 16), jnp.float32),
            ],
        )
    return _EDGE_KERNEL


def _gat(x, prep, p):
    W, a_s, a_d, b = p
    srcs, dls = prep
    k = W.shape[0]
    w_ext = jnp.concatenate(
        [W, (W @ a_s)[:, None], (W @ a_d)[:, None],
         jnp.zeros((k, 126), jnp.float32)], axis=1)
    xe = _mm(x, w_ext)
    xq = [xe[:, 128 * i:128 * (i + 1)] for i in range(4)]
    pad = jnp.zeros((16,), jnp.float32)
    s_p = jnp.concatenate([xe[:, 512], pad])
    d_p = jnp.concatenate([xe[:, 513], pad])
    oq = _edge_kernel()(xq[0], xq[1], xq[2], xq[3], s_p, d_p, srcs, dls)
    top = jnp.concatenate([o[:_HALF] for o in oq], axis=1)
    bot = jnp.concatenate([o[_ROWS:_ROWS + _HALF] for o in oq], axis=1)
    return jnp.concatenate([top, bot], axis=0) + b


def _lin(x, p, rows=1000):
    w, b = p
    n = w.shape[1]
    if n % 128 != 0:
        w = jnp.concatenate(
            [w, jnp.zeros((w.shape[0], 128 - n % 128), jnp.float32)],
            axis=1)
        return _mm(x, w, rows)[:, :n] + b
    return _mm(x, w, rows) + b


def kernel(x, edge_index_1, edge_index_2, edge_index_n, num_graphs,
           conv0, conv1_1, conv1_2, conv2, conv3, conv4,
           lin1, lin2, lin3, lin4):
    n = x.shape[0]
    npg = n // _NG
    prep1 = _edge_prep(edge_index_1)
    prep2 = _edge_prep(edge_index_2)
    prepn = _edge_prep(edge_index_n)
    x = x * (jnp.asarray(num_graphs, dtype=x.dtype) / jnp.float32(_NG))
    out = _gat(x, prepn, conv0)
    out_1 = _gat(out, prep1, conv1_1)
    out_2 = _gat(out, prep2, conv1_2)
    out_lin1 = _lin(out_1 + out_2, lin1)
    out_lin2 = _lin(out_lin1, lin2)
    out_lin3 = _lin(out_lin2, lin3) + out_lin1
    left = _gat(out_lin3, prep1, conv2)
    left = _gat(left, prep1, conv3)
    left = _gat(left, prepn, conv4)
    left = jnp.squeeze(_lin(left, lin4).reshape(_NG, npg, -1), axis=-1)
    probs_left = jax.nn.softmax(left, axis=-1)
    right = _gat(out_lin3, prep2, conv2)
    right = _gat(right, prep2, conv3)
    right = _gat(right, prepn, conv4)
    right = jnp.squeeze(_lin(right, lin4).reshape(_NG, npg, -1), axis=-1)
    probs_right = jax.nn.softmax(right, axis=-1)
    out_to_critic = out_lin3.reshape(_NG, npg, -1)
    return (probs_left, probs_right, out_to_critic)


# speedup vs baseline: 3.4363x; 1.0632x over previous
import functools

import jax
import jax.numpy as jnp
from jax import lax
from jax.experimental import pallas as pl
from jax.experimental.pallas import tpu as pltpu
from jax.experimental.pallas import tpu_sc as plsc

_N = 10000
_NG = 10
_E = 160000
_EL = _E + _N
_HALF = _N // 2
_P = 90112
_TCH = 88
_TE = _TCH * 64
_ROWS = 5120
_STRIPE = _ROWS // 16
_SCRATCH = 5000


def _mm_kernel(x_ref, w_ref, o_ref):
    o_ref[...] = jnp.dot(x_ref[...], w_ref[...],
                         preferred_element_type=jnp.float32)


def _mm(x, w, rows=1000):
    m, k = x.shape
    n = w.shape[1]
    return pl.pallas_call(
        _mm_kernel,
        grid=(m // rows,),
        in_specs=[pl.BlockSpec((rows, k), lambda i: (i, 0)),
                  pl.BlockSpec((k, n), lambda i: (0, 0))],
        out_specs=pl.BlockSpec((rows, n), lambda i: (i, 0)),
        out_shape=jax.ShapeDtypeStruct((m, n), jnp.float32),
    )(x, w)


def _edge_prep(ei):
    loop = jnp.arange(_N, dtype=ei.dtype)
    src = jnp.concatenate([ei[0], loop]).astype(jnp.int32)
    dst = jnp.concatenate([ei[1], loop]).astype(jnp.int32)
    side = (dst >= _HALF).astype(jnp.int32)
    perm = jnp.argsort(side, stable=True).astype(jnp.int32)
    p0 = perm[:_P]
    p1 = perm[_EL - _P:]
    v0 = side[p0] == 0
    v1 = side[p1] == 1
    src0 = jnp.where(v0, src[p0], 0)
    dl0 = jnp.where(v0, dst[p0], _SCRATCH)
    src1 = jnp.where(v1, src[p1], 0)
    dl1 = jnp.where(v1, dst[p1] - _HALF, _SCRATCH)
    srcs = jnp.stack([src0, src1]).reshape(2, _P // 64, 64)
    dls = jnp.stack([dl0, dl1]).reshape(2, _P // 64, 64)
    return srcs, dls


def _edge_body(xpq0, xpq1, xpq2, xpq3, s_hbm, d_hbm, srcs_hbm, dls_hbm,
               o0_hbm, o1_hbm, o2_hbm, o3_hbm,
               s_v, d_v, srcc_v, dlc_v, e_v, rows_v, rows2_v, den_v,
               red_v, z_v, m16_v, gsem, gsem2, out_s, den_s, mst_s):
    c = lax.axis_index("c")
    sid = lax.axis_index("s")
    coff = c * _HALF

    pltpu.sync_copy(s_hbm, s_v)
    pltpu.sync_copy(d_hbm, d_v)
    pltpu.sync_copy(srcs_hbm.at[c, pl.ds(sid * _TCH, _TCH)], srcc_v)
    pltpu.sync_copy(dls_hbm.at[c, pl.ds(sid * _TCH, _TCH)], dlc_v)

    def p1_body(j, mx):
        def p1_inner(k, mx):
            si = srcc_v[j, pl.ds(k * 16, 16)]
            di = dlc_v[j, pl.ds(k * 16, 16)] + coff
            z = plsc.load_gather(s_v, [si]) + plsc.load_gather(d_v, [di])
            e = jnp.where(z > 0, z, 0.2 * z)
            e_v[pl.ds(j * 64 + k * 16, 16)] = e
            return jnp.maximum(mx, e)
        mx = p1_inner(0, mx)
        mx = p1_inner(1, mx)
        mx = p1_inner(2, mx)
        mx = p1_inner(3, mx)
        return mx
    mx = lax.fori_loop(0, _TCH, p1_body,
                       jnp.full((16,), -3e38, jnp.float32))
    red_v[...] = mx

    pltpu.sync_copy(red_v, mst_s.at[sid])
    plsc.subcore_barrier()
    def z_body(i, _):
        z_v[pl.ds(i * 16, 16)] = jnp.zeros((16,), jnp.float32)
        return 0
    lax.fori_loop(0, _STRIPE // 16, z_body, 0)
    pltpu.sync_copy(z_v, den_s.at[pl.ds(sid * _STRIPE, _STRIPE)])
    pltpu.sync_copy(mst_s, m16_v)
    mall = m16_v[0, pl.ds(0, 16)]
    for t in range(1, 16):
        mall = jnp.maximum(mall, m16_v[t, pl.ds(0, 16)])
    m = jnp.max(mall)
    mb = jnp.full((16,), m)

    def p2_body(i, _):
        e_v[pl.ds(i * 16, 16)] = jnp.exp(e_v[pl.ds(i * 16, 16)] - mb)
        return 0
    lax.fori_loop(0, _TE // 16, p2_body, 0)
    plsc.subcore_barrier()

    def den_body(j, _):
        pltpu.sync_copy(e_v.at[pl.ds(j * 64, 64)],
                        den_s.at[dlc_v.at[j]], add=True)
        return 0
    lax.fori_loop(0, _TCH, den_body, 0)
    plsc.subcore_barrier()
    pltpu.sync_copy(den_s, den_v)

    def p3_body(j, _):
        def p3_inner(k):
            dl = dlc_v[j, pl.ds(k * 16, 16)]
            dn = plsc.load_gather(den_v, [dl])
            sl = pl.ds(j * 64 + k * 16, 16)
            e_v[sl] = e_v[sl] / (dn + 1e-16)
        p3_inner(0)
        p3_inner(1)
        p3_inner(2)
        p3_inner(3)
        return 0
    lax.fori_loop(0, _TCH, p3_body, 0)

    for xp_hbm, o_hbm in ((xpq0, o0_hbm), (xpq1, o1_hbm),
                          (xpq2, o2_hbm), (xpq3, o3_hbm)):
        def zr_body(r, _):
            for ck in range(8):
                rows_v[r, pl.ds(ck * 16, 16)] = jnp.zeros((16,),
                                                          jnp.float32)
            return 0
        lax.fori_loop(0, 64, zr_body, 0)
        for q in range(_STRIPE // 64):
            pltpu.sync_copy(rows_v,
                            out_s.at[pl.ds(sid * _STRIPE + q * 64, 64)])
        plsc.subcore_barrier()

        def scale_scatter(buf, j):
            def row_body(r4, _):
                for u in range(4):
                    r = r4 * 4 + u
                    av = plsc.load_gather(
                        e_v, [jnp.full((16,), j * 64 + r, jnp.int32)])
                    for ck in range(8):
                        sl = pl.ds(ck * 16, 16)
                        buf[r, sl] = buf[r, sl] * av
                return 0
            lax.fori_loop(0, 16, row_body, 0)
            pltpu.sync_copy(buf, out_s.at[dlc_v.at[j]], add=True)

        pltpu.async_copy(xp_hbm.at[srcc_v.at[0]], rows_v, gsem)

        def pair_body(j2, _):
            j = j2 * 2
            pltpu.make_async_copy(xp_hbm.at[srcc_v.at[j]],
                                  rows_v, gsem).wait()
            pltpu.async_copy(xp_hbm.at[srcc_v.at[j + 1]], rows2_v, gsem2)
            scale_scatter(rows_v, j)
            pltpu.make_async_copy(xp_hbm.at[srcc_v.at[j + 1]],
                                  rows2_v, gsem2).wait()

            @pl.when(j + 2 < _TCH)
            def _():
                pltpu.async_copy(xp_hbm.at[srcc_v.at[j + 2]],
                                 rows_v, gsem)
            scale_scatter(rows2_v, j + 1)
            return 0
        lax.fori_loop(0, _TCH // 2, pair_body, 0)
        plsc.subcore_barrier()

        for q in range(_STRIPE // 64):
            pltpu.sync_copy(out_s.at[pl.ds(sid * _STRIPE + q * 64, 64)],
                            rows_v)
            pltpu.sync_copy(
                rows_v,
                o_hbm.at[pl.ds(c * _ROWS + sid * _STRIPE + q * 64, 64)])
        plsc.subcore_barrier()


_EDGE_KERNEL = None


def _edge_kernel():
    global _EDGE_KERNEL
    if _EDGE_KERNEL is None:
        mesh = plsc.VectorSubcoreMesh(core_axis_name="c",
                                      subcore_axis_name="s")
        _EDGE_KERNEL = pl.kernel(
            _edge_body,
            mesh=mesh,
            compiler_params=pltpu.CompilerParams(
                needs_layout_passes=False),
            out_type=[
                jax.ShapeDtypeStruct((2 * _ROWS, 128), jnp.float32)
                for _ in range(4)
            ],
            scratch_types=[
                pltpu.VMEM((_N + 16,), jnp.float32),
                pltpu.VMEM((_N + 16,), jnp.float32),
                pltpu.VMEM((_TCH, 64), jnp.int32),
                pltpu.VMEM((_TCH, 64), jnp.int32),
                pltpu.VMEM((_TE,), jnp.float32),
                pltpu.VMEM((64, 128), jnp.float32),
                pltpu.VMEM((64, 128), jnp.float32),
                pltpu.VMEM((_ROWS,), jnp.float32),
                pltpu.VMEM((16,), jnp.float32),
                pltpu.VMEM((_STRIPE,), jnp.float32),
                pltpu.VMEM((16, 16), jnp.float32),
                pltpu.SemaphoreType.DMA,
                pltpu.SemaphoreType.DMA,
                pltpu.VMEM_SHARED((_ROWS, 128), jnp.float32),
                pltpu.VMEM_SHARED((_ROWS,), jnp.float32),
                pltpu.VMEM_SHARED((16, 16), jnp.float32),
            ],
        )
    return _EDGE_KERNEL


def _gat(x, prep, p):
    W, a_s, a_d, b = p
    srcs, dls = prep
    k = W.shape[0]
    w_ext = jnp.concatenate(
        [W, (W @ a_s)[:, None], (W @ a_d)[:, None],
         jnp.zeros((k, 126), jnp.float32)], axis=1)
    xe = _mm(x, w_ext)
    xq = [xe[:, 128 * i:128 * (i + 1)] for i in range(4)]
    pad = jnp.zeros((16,), jnp.float32)
    s_p = jnp.concatenate([xe[:, 512], pad])
    d_p = jnp.concatenate([xe[:, 513], pad])
    oq = _edge_kernel()(xq[0], xq[1], xq[2], xq[3], s_p, d_p, srcs, dls)
    top = jnp.concatenate([o[:_HALF] for o in oq], axis=1)
    bot = jnp.concatenate([o[_ROWS:_ROWS + _HALF] for o in oq], axis=1)
    return jnp.concatenate([top, bot], axis=0) + b


def _lin(x, p, rows=1000):
    w, b = p
    n = w.shape[1]
    if n % 128 != 0:
        w = jnp.concatenate(
            [w, jnp.zeros((w.shape[0], 128 - n % 128), jnp.float32)],
            axis=1)
        return _mm(x, w, rows)[:, :n] + b
    return _mm(x, w, rows) + b


def kernel(x, edge_index_1, edge_index_2, edge_index_n, num_graphs,
           conv0, conv1_1, conv1_2, conv2, conv3, conv4,
           lin1, lin2, lin3, lin4):
    n = x.shape[0]
    npg = n // _NG
    prep1 = _edge_prep(edge_index_1)
    prep2 = _edge_prep(edge_index_2)
    prepn = _edge_prep(edge_index_n)
    x = x * (jnp.asarray(num_graphs, dtype=x.dtype) / jnp.float32(_NG))
    out = _gat(x, prepn, conv0)
    out_1 = _gat(out, prep1, conv1_1)
    out_2 = _gat(out, prep2, conv1_2)
    out_lin1 = _lin(out_1 + out_2, lin1)
    out_lin2 = _lin(out_lin1, lin2)
    out_lin3 = _lin(out_lin2, lin3) + out_lin1
    left = _gat(out_lin3, prep1, conv2)
    left = _gat(left, prep1, conv3)
    left = _gat(left, prepn, conv4)
    left = jnp.squeeze(_lin(left, lin4).reshape(_NG, npg, -1), axis=-1)
    probs_left = jax.nn.softmax(left, axis=-1)
    right = _gat(out_lin3, prep2, conv2)
    right = _gat(right, prep2, conv3)
    right = _gat(right, prepn, conv4)
    right = jnp.squeeze(_lin(right, lin4).reshape(_NG, npg, -1), axis=-1)
    probs_right = jax.nn.softmax(right, axis=-1)
    out_to_critic = out_lin3.reshape(_NG, npg, -1)
    return (probs_left, probs_right, out_to_critic)


# speedup vs baseline: 3.4569x; 1.0060x over previous
import functools

import jax
import jax.numpy as jnp
from jax import lax
from jax.experimental import pallas as pl
from jax.experimental.pallas import tpu as pltpu
from jax.experimental.pallas import tpu_sc as plsc

_N = 10000
_NG = 10
_E = 160000
_EL = _E + _N
_HALF = _N // 2
_P = 90112
_TCH = 88
_TE = _TCH * 64
_ROWS = 5120
_STRIPE = _ROWS // 16
_SCRATCH = 5000


def _mm_kernel(x_ref, w_ref, o_ref):
    o_ref[...] = jnp.dot(x_ref[...], w_ref[...],
                         preferred_element_type=jnp.float32)


def _mm(x, w, rows=1000):
    m, k = x.shape
    n = w.shape[1]
    return pl.pallas_call(
        _mm_kernel,
        grid=(m // rows,),
        in_specs=[pl.BlockSpec((rows, k), lambda i: (i, 0)),
                  pl.BlockSpec((k, n), lambda i: (0, 0))],
        out_specs=pl.BlockSpec((rows, n), lambda i: (i, 0)),
        out_shape=jax.ShapeDtypeStruct((m, n), jnp.float32),
    )(x, w)


def _edge_prep(ei):
    loop = jnp.arange(_N, dtype=ei.dtype)
    src = jnp.concatenate([ei[0], loop]).astype(jnp.int32)
    dst = jnp.concatenate([ei[1], loop]).astype(jnp.int32)
    side = (dst >= _HALF).astype(jnp.int32)
    perm = jnp.argsort(side, stable=True).astype(jnp.int32)
    p0 = perm[:_P]
    p1 = perm[_EL - _P:]
    v0 = side[p0] == 0
    v1 = side[p1] == 1
    src0 = jnp.where(v0, src[p0], 0)
    dl0 = jnp.where(v0, dst[p0], _SCRATCH)
    src1 = jnp.where(v1, src[p1], 0)
    dl1 = jnp.where(v1, dst[p1] - _HALF, _SCRATCH)
    srcs = jnp.stack([src0, src1]).reshape(2, _P // 64, 64)
    dls = jnp.stack([dl0, dl1]).reshape(2, _P // 64, 64)
    return srcs, dls


def _edge_body(xpq0, xpq1, xpq2, xpq3, s_hbm, d_hbm, srcs_hbm, dls_hbm,
               o0_hbm, o1_hbm, o2_hbm, o3_hbm,
               s_v, d_v, srcc_v, dlc_v, e_v, rows_v, rows2_v, den_v,
               red_v, z_v, m16_v, gsem, gsem2, out_s, den_s, mst_s):
    c = lax.axis_index("c")
    sid = lax.axis_index("s")
    coff = c * _HALF

    pltpu.sync_copy(s_hbm, s_v)
    pltpu.sync_copy(d_hbm, d_v)
    pltpu.sync_copy(srcs_hbm.at[c, pl.ds(sid * _TCH, _TCH)], srcc_v)
    pltpu.sync_copy(dls_hbm.at[c, pl.ds(sid * _TCH, _TCH)], dlc_v)

    def p1_body(j, mx):
        def p1_inner(k, mx):
            si = srcc_v[j, pl.ds(k * 16, 16)]
            di = dlc_v[j, pl.ds(k * 16, 16)] + coff
            z = plsc.load_gather(s_v, [si]) + plsc.load_gather(d_v, [di])
            e = jnp.where(z > 0, z, 0.2 * z)
            e_v[pl.ds(j * 64 + k * 16, 16)] = e
            return jnp.maximum(mx, e)
        mx = p1_inner(0, mx)
        mx = p1_inner(1, mx)
        mx = p1_inner(2, mx)
        mx = p1_inner(3, mx)
        return mx
    mx = plsc.parallel_loop(
        0, _TCH, unroll=2,
        carry=jnp.full((16,), -3e38, jnp.float32))(p1_body)
    red_v[...] = mx

    pltpu.sync_copy(red_v, mst_s.at[sid])
    plsc.subcore_barrier()
    def z_body(i, _):
        z_v[pl.ds(i * 16, 16)] = jnp.zeros((16,), jnp.float32)
        return 0
    lax.fori_loop(0, _STRIPE // 16, z_body, 0)
    pltpu.sync_copy(z_v, den_s.at[pl.ds(sid * _STRIPE, _STRIPE)])
    pltpu.sync_copy(mst_s, m16_v)
    mall = m16_v[0, pl.ds(0, 16)]
    for t in range(1, 16):
        mall = jnp.maximum(mall, m16_v[t, pl.ds(0, 16)])
    m = jnp.max(mall)
    mb = jnp.full((16,), m)

    @plsc.parallel_loop(0, _TE // 16, unroll=4)
    def _(i):
        e_v[pl.ds(i * 16, 16)] = jnp.exp(e_v[pl.ds(i * 16, 16)] - mb)
    plsc.subcore_barrier()

    def den_body(j, _):
        pltpu.sync_copy(e_v.at[pl.ds(j * 64, 64)],
                        den_s.at[dlc_v.at[j]], add=True)
        return 0
    lax.fori_loop(0, _TCH, den_body, 0)
    plsc.subcore_barrier()
    pltpu.sync_copy(den_s, den_v)

    @plsc.parallel_loop(0, _TCH, unroll=2)
    def _(j):
        for k in range(4):
            dl = dlc_v[j, pl.ds(k * 16, 16)]
            dn = plsc.load_gather(den_v, [dl])
            sl = pl.ds(j * 64 + k * 16, 16)
            e_v[sl] = e_v[sl] / (dn + 1e-16)

    for xp_hbm, o_hbm in ((xpq0, o0_hbm), (xpq1, o1_hbm),
                          (xpq2, o2_hbm), (xpq3, o3_hbm)):
        def zr_body(r, _):
            for ck in range(8):
                rows_v[r, pl.ds(ck * 16, 16)] = jnp.zeros((16,),
                                                          jnp.float32)
            return 0
        lax.fori_loop(0, 64, zr_body, 0)
        for q in range(_STRIPE // 64):
            pltpu.sync_copy(rows_v,
                            out_s.at[pl.ds(sid * _STRIPE + q * 64, 64)])
        plsc.subcore_barrier()

        def scale_scatter(buf, j):
            @plsc.parallel_loop(0, 64, unroll=4)
            def _(r):
                av = plsc.load_gather(
                    e_v, [jnp.full((16,), j * 64 + r, jnp.int32)])
                for ck in range(8):
                    sl = pl.ds(ck * 16, 16)
                    buf[r, sl] = buf[r, sl] * av
            pltpu.sync_copy(buf, out_s.at[dlc_v.at[j]], add=True)

        pltpu.async_copy(xp_hbm.at[srcc_v.at[0]], rows_v, gsem)

        def pair_body(j2, _):
            j = j2 * 2
            pltpu.make_async_copy(xp_hbm.at[srcc_v.at[j]],
                                  rows_v, gsem).wait()
            pltpu.async_copy(xp_hbm.at[srcc_v.at[j + 1]], rows2_v, gsem2)
            scale_scatter(rows_v, j)
            pltpu.make_async_copy(xp_hbm.at[srcc_v.at[j + 1]],
                                  rows2_v, gsem2).wait()

            @pl.when(j + 2 < _TCH)
            def _():
                pltpu.async_copy(xp_hbm.at[srcc_v.at[j + 2]],
                                 rows_v, gsem)
            scale_scatter(rows2_v, j + 1)
            return 0
        lax.fori_loop(0, _TCH // 2, pair_body, 0)
        plsc.subcore_barrier()

        for q in range(_STRIPE // 64):
            pltpu.sync_copy(out_s.at[pl.ds(sid * _STRIPE + q * 64, 64)],
                            rows_v)
            pltpu.sync_copy(
                rows_v,
                o_hbm.at[pl.ds(c * _ROWS + sid * _STRIPE + q * 64, 64)])
        plsc.subcore_barrier()


_EDGE_KERNEL = None


def _edge_kernel():
    global _EDGE_KERNEL
    if _EDGE_KERNEL is None:
        mesh = plsc.VectorSubcoreMesh(core_axis_name="c",
                                      subcore_axis_name="s")
        _EDGE_KERNEL = pl.kernel(
            _edge_body,
            mesh=mesh,
            compiler_params=pltpu.CompilerParams(
                needs_layout_passes=False),
            out_type=[
                jax.ShapeDtypeStruct((2 * _ROWS, 128), jnp.float32)
                for _ in range(4)
            ],
            scratch_types=[
                pltpu.VMEM((_N + 16,), jnp.float32),
                pltpu.VMEM((_N + 16,), jnp.float32),
                pltpu.VMEM((_TCH, 64), jnp.int32),
                pltpu.VMEM((_TCH, 64), jnp.int32),
                pltpu.VMEM((_TE,), jnp.float32),
                pltpu.VMEM((64, 128), jnp.float32),
                pltpu.VMEM((64, 128), jnp.float32),
                pltpu.VMEM((_ROWS,), jnp.float32),
                pltpu.VMEM((16,), jnp.float32),
                pltpu.VMEM((_STRIPE,), jnp.float32),
                pltpu.VMEM((16, 16), jnp.float32),
                pltpu.SemaphoreType.DMA,
                pltpu.SemaphoreType.DMA,
                pltpu.VMEM_SHARED((_ROWS, 128), jnp.float32),
                pltpu.VMEM_SHARED((_ROWS,), jnp.float32),
                pltpu.VMEM_SHARED((16, 16), jnp.float32),
            ],
        )
    return _EDGE_KERNEL


def _gat(x, prep, p):
    W, a_s, a_d, b = p
    srcs, dls = prep
    k = W.shape[0]
    w_ext = jnp.concatenate(
        [W, (W @ a_s)[:, None], (W @ a_d)[:, None],
         jnp.zeros((k, 126), jnp.float32)], axis=1)
    xe = _mm(x, w_ext)
    xq = [xe[:, 128 * i:128 * (i + 1)] for i in range(4)]
    pad = jnp.zeros((16,), jnp.float32)
    s_p = jnp.concatenate([xe[:, 512], pad])
    d_p = jnp.concatenate([xe[:, 513], pad])
    oq = _edge_kernel()(xq[0], xq[1], xq[2], xq[3], s_p, d_p, srcs, dls)
    top = jnp.concatenate([o[:_HALF] for o in oq], axis=1)
    bot = jnp.concatenate([o[_ROWS:_ROWS + _HALF] for o in oq], axis=1)
    return jnp.concatenate([top, bot], axis=0) + b


def _lin(x, p, rows=1000):
    w, b = p
    n = w.shape[1]
    if n % 128 != 0:
        w = jnp.concatenate(
            [w, jnp.zeros((w.shape[0], 128 - n % 128), jnp.float32)],
            axis=1)
        return _mm(x, w, rows)[:, :n] + b
    return _mm(x, w, rows) + b


def kernel(x, edge_index_1, edge_index_2, edge_index_n, num_graphs,
           conv0, conv1_1, conv1_2, conv2, conv3, conv4,
           lin1, lin2, lin3, lin4):
    n = x.shape[0]
    npg = n // _NG
    prep1 = _edge_prep(edge_index_1)
    prep2 = _edge_prep(edge_index_2)
    prepn = _edge_prep(edge_index_n)
    x = x * (jnp.asarray(num_graphs, dtype=x.dtype) / jnp.float32(_NG))
    out = _gat(x, prepn, conv0)
    out_1 = _gat(out, prep1, conv1_1)
    out_2 = _gat(out, prep2, conv1_2)
    out_lin1 = _lin(out_1 + out_2, lin1)
    out_lin2 = _lin(out_lin1, lin2)
    out_lin3 = _lin(out_lin2, lin3) + out_lin1
    left = _gat(out_lin3, prep1, conv2)
    left = _gat(left, prep1, conv3)
    left = _gat(left, prepn, conv4)
    left = jnp.squeeze(_lin(left, lin4).reshape(_NG, npg, -1), axis=-1)
    probs_left = jax.nn.softmax(left, axis=-1)
    right = _gat(out_lin3, prep2, conv2)
    right = _gat(right, prep2, conv3)
    right = _gat(right, prepn, conv4)
    right = jnp.squeeze(_lin(right, lin4).reshape(_NG, npg, -1), axis=-1)
    probs_right = jax.nn.softmax(right, axis=-1)
    out_to_critic = out_lin3.reshape(_NG, npg, -1)
    return (probs_left, probs_right, out_to_critic)
